# Initial kernel scaffold; baseline (speedup 1.0000x reference)
import jax
import jax.numpy as jnp
from jax.experimental import pallas as pl


def kernel(x, init_x, edge_index, W, b):
    raise NotImplementedError("write your pallas kernel here")


# speedup vs baseline: 7.7853x; 7.7853x over previous
import functools

import jax
import jax.numpy as jnp
from jax import lax
from jax.experimental import pallas as pl
from jax.experimental.pallas import tpu as pltpu
from jax.experimental.pallas import tpu_sc as plsc

N_NODES = 10000
N_EDGES = 320000
DIM = 128
ALPHA = 0.1
BETA = 1.0

NC = 2
NS = 16
NW = NC * NS
EW = N_EDGES // NW
CHUNK = 80
NCHUNK = EW // CHUNK
NPAD = 10240
ROWS_PER_TILE = NPAD // NS
ZCHUNK = 80
WCHUNK = 128
LANES = 16

_MESH = plsc.VectorSubcoreMesh(
    core_axis_name="c", subcore_axis_name="s", num_cores=NC, num_subcores=NS
)


@functools.partial(
    pl.kernel,
    out_type=jax.ShapeDtypeStruct((NC, NPAD, DIM), jnp.float32),
    mesh=_MESH,
    scratch_types=[
        pltpu.VMEM((NCHUNK, CHUNK), jnp.int32),
        pltpu.VMEM((NCHUNK, CHUNK), jnp.int32),
        pltpu.VMEM((CHUNK, DIM), jnp.float32),
        pltpu.VMEM_SHARED((NPAD, DIM), jnp.float32),
        pltpu.SemaphoreType.DMA,
    ],
)
def _spmm_sc(x_hbm, src_hbm, dst_hbm, part_hbm, src_v, dst_v, rows_v, acc_sh,
             sem):
    c = lax.axis_index("c")
    s = lax.axis_index("s")
    wid = s * NC + c

    def _zfill(t, carry):
        i = t // (DIM // LANES)
        k = t % (DIM // LANES)
        rows_v[i, pl.ds(k * LANES, LANES)] = jnp.zeros((LANES,), jnp.float32)
        return carry

    lax.fori_loop(0, ZCHUNK * (DIM // LANES), _zfill, 0)

    def _zcopy(t, carry):
        off = pl.multiple_of(s * ROWS_PER_TILE + t * ZCHUNK, 8)
        pltpu.sync_copy(rows_v, acc_sh.at[pl.ds(off, ZCHUNK)])
        return carry

    lax.fori_loop(0, ROWS_PER_TILE // ZCHUNK, _zcopy, 0)
    plsc.subcore_barrier()

    pltpu.sync_copy(src_hbm.at[wid], src_v)
    pltpu.sync_copy(dst_hbm.at[wid], dst_v)

    def _edge_chunk(j, carry):
        pltpu.async_copy(x_hbm.at[src_v.at[j]], rows_v, sem).wait()
        pltpu.sync_copy(rows_v, acc_sh.at[dst_v.at[j]], add=True)
        return carry

    lax.fori_loop(0, NCHUNK, _edge_chunk, 0)
    plsc.subcore_barrier()

    def _wback(t, carry):
        off = pl.multiple_of(s * ROWS_PER_TILE + t * WCHUNK, 8)
        pltpu.sync_copy(acc_sh.at[pl.ds(off, WCHUNK)],
                        part_hbm.at[c, pl.ds(off, WCHUNK)])
        return carry

    lax.fori_loop(0, ROWS_PER_TILE // WCHUNK, _wback, 0)


RBLK = 2000


def _mix_mm_tc(part_ref, init_ref, w_ref, b_ref, out_ref):
    h = (1.0 - ALPHA) * (part_ref[0] + part_ref[1]) + ALPHA * init_ref[...]
    mm = lax.dot_general(h, w_ref[...], (((1,), (1,)), ((), ())),
                         preferred_element_type=jnp.float32)
    out_ref[...] = BETA * (mm + b_ref[...]) + (1.0 - BETA) * h


def kernel(x, init_x, edge_index, W, b):
    src = edge_index[0].reshape(NW, NCHUNK, CHUNK)
    dst = edge_index[1].reshape(NW, NCHUNK, CHUNK)
    part = _spmm_sc(x, src, dst)
    out = pl.pallas_call(
        _mix_mm_tc,
        grid=(N_NODES // RBLK,),
        in_specs=[
            pl.BlockSpec((NC, RBLK, DIM), lambda i: (0, i, 0)),
            pl.BlockSpec((RBLK, DIM), lambda i: (i, 0)),
            pl.BlockSpec((DIM, DIM), lambda i: (0, 0)),
            pl.BlockSpec((1, DIM), lambda i: (0, 0)),
        ],
        out_specs=pl.BlockSpec((RBLK, DIM), lambda i: (i, 0)),
        out_shape=jax.ShapeDtypeStruct((N_NODES, DIM), jnp.float32),
    )(part, init_x, W, b.reshape(1, DIM))
    return out
